# initial kernel scaffold (unmeasured)
import jax
import jax.numpy as jnp
from jax import lax
from jax.experimental import pallas as pl
from jax.experimental.pallas import tpu as pltpu


def kernel(
    x,
):
    def body(*refs):
        pass

    out_shape = jax.ShapeDtypeStruct(..., jnp.float32)
    return pl.pallas_call(body, out_shape=out_shape)(...)


# baseline (device time: 1106013 ns/iter reference)
import jax
import jax.numpy as jnp
from jax import lax
from jax.experimental import pallas as pl
from jax.experimental.pallas import tpu as pltpu

N_DEV = 4


def kernel(x):
    m_per, n = x.shape
    m_half = m_per // 2

    def body(x_ref, out_ref, local_sem, cw_send, cw_recv, ccw_send, ccw_recv):
        my = lax.axis_index("i")
        left = (my - 1) % N_DEV
        right = (my + 1) % N_DEV

        barrier = pltpu.get_barrier_semaphore()
        for nbr in (left, right):
            pl.semaphore_signal(
                barrier, inc=1,
                device_id=(nbr,), device_id_type=pl.DeviceIdType.MESH,
            )
        pl.semaphore_wait(barrier, 2)

        local = pltpu.make_async_copy(
            x_ref, out_ref.at[pl.ds(my * m_per, m_per)], local_sem
        )
        local.start()

        for h in range(N_DEV - 1):
            o_cw = (my - h) % N_DEV
            o_ccw = (my + h) % N_DEV
            if h == 0:
                cw_src = x_ref.at[pl.ds(0, m_half)]
                ccw_src = x_ref.at[pl.ds(m_half, m_half)]
            else:
                cw_src = out_ref.at[pl.ds(o_cw * m_per, m_half)]
                ccw_src = out_ref.at[pl.ds(o_ccw * m_per + m_half, m_half)]

            cw = pltpu.make_async_remote_copy(
                src_ref=cw_src,
                dst_ref=out_ref.at[pl.ds(o_cw * m_per, m_half)],
                send_sem=cw_send.at[h],
                recv_sem=cw_recv.at[h],
                device_id=(right,),
                device_id_type=pl.DeviceIdType.MESH,
            )
            ccw = pltpu.make_async_remote_copy(
                src_ref=ccw_src,
                dst_ref=out_ref.at[pl.ds(o_ccw * m_per + m_half, m_half)],
                send_sem=ccw_send.at[h],
                recv_sem=ccw_recv.at[h],
                device_id=(left,),
                device_id_type=pl.DeviceIdType.MESH,
            )
            cw.start()
            ccw.start()
            cw.wait()
            ccw.wait()

        local.wait()

    return pl.pallas_call(
        body,
        out_shape=jax.ShapeDtypeStruct((N_DEV * m_per, n), x.dtype),
        in_specs=[pl.BlockSpec(memory_space=pl.ANY)],
        out_specs=pl.BlockSpec(memory_space=pl.ANY),
        scratch_shapes=[
            pltpu.SemaphoreType.DMA,
            pltpu.SemaphoreType.DMA((N_DEV - 1,)),
            pltpu.SemaphoreType.DMA((N_DEV - 1,)),
            pltpu.SemaphoreType.DMA((N_DEV - 1,)),
            pltpu.SemaphoreType.DMA((N_DEV - 1,)),
        ],
        compiler_params=pltpu.CompilerParams(collective_id=0),
    )(x)


# device time: 635077 ns/iter; 1.7415x vs baseline; 1.7415x over previous
import jax
import jax.numpy as jnp
from jax import lax
from jax.experimental import pallas as pl
from jax.experimental.pallas import tpu as pltpu

N_DEV = 4


def kernel(x):
    m_per, n = x.shape
    m_half = m_per // 2

    rows = 1024
    nblk = m_per // rows

    def body(x_ref, out_ref, vbuf, ld_sems, st_sems,
             cw_send, cw_recv, ccw_send, ccw_recv):
        my = lax.axis_index("i")
        left = (my - 1) % N_DEV
        right = (my + 1) % N_DEV

        barrier = pltpu.get_barrier_semaphore()
        for nbr in (left, right):
            pl.semaphore_signal(
                barrier, inc=1,
                device_id=(nbr,), device_id_type=pl.DeviceIdType.MESH,
            )
        pl.semaphore_wait(barrier, 2)

        def local_copy():
            def load(b):
                return pltpu.make_async_copy(
                    x_ref.at[pl.ds(b * rows, rows)], vbuf.at[b % 2],
                    ld_sems.at[b % 2])

            def store(b):
                return pltpu.make_async_copy(
                    vbuf.at[b % 2], out_ref.at[pl.ds(my * m_per + b * rows, rows)],
                    st_sems.at[b % 2])

            load(0).start()
            for b in range(nblk):
                load(b).wait()
                store(b).start()
                if b + 1 < nblk:
                    if b >= 1:
                        store(b - 1).wait()
                    load(b + 1).start()
            store(nblk - 2).wait()
            store(nblk - 1).wait()

        for h in range(N_DEV - 1):
            o_cw = (my - h) % N_DEV
            o_ccw = (my + h) % N_DEV
            if h == 0:
                cw_src = x_ref.at[pl.ds(0, m_half)]
                ccw_src = x_ref.at[pl.ds(m_half, m_half)]
            else:
                cw_src = out_ref.at[pl.ds(o_cw * m_per, m_half)]
                ccw_src = out_ref.at[pl.ds(o_ccw * m_per + m_half, m_half)]

            cw = pltpu.make_async_remote_copy(
                src_ref=cw_src,
                dst_ref=out_ref.at[pl.ds(o_cw * m_per, m_half)],
                send_sem=cw_send.at[h],
                recv_sem=cw_recv.at[h],
                device_id=(right,),
                device_id_type=pl.DeviceIdType.MESH,
            )
            ccw = pltpu.make_async_remote_copy(
                src_ref=ccw_src,
                dst_ref=out_ref.at[pl.ds(o_ccw * m_per + m_half, m_half)],
                send_sem=ccw_send.at[h],
                recv_sem=ccw_recv.at[h],
                device_id=(left,),
                device_id_type=pl.DeviceIdType.MESH,
            )
            cw.start()
            ccw.start()
            if h == 0:
                local_copy()
            cw.wait()
            ccw.wait()

    return pl.pallas_call(
        body,
        out_shape=jax.ShapeDtypeStruct((N_DEV * m_per, n), x.dtype),
        in_specs=[pl.BlockSpec(memory_space=pl.ANY)],
        out_specs=pl.BlockSpec(memory_space=pl.ANY),
        scratch_shapes=[
            pltpu.VMEM((2, rows, n), jnp.float32),
            pltpu.SemaphoreType.DMA((2,)),
            pltpu.SemaphoreType.DMA((2,)),
            pltpu.SemaphoreType.DMA((N_DEV - 1,)),
            pltpu.SemaphoreType.DMA((N_DEV - 1,)),
            pltpu.SemaphoreType.DMA((N_DEV - 1,)),
            pltpu.SemaphoreType.DMA((N_DEV - 1,)),
        ],
        compiler_params=pltpu.CompilerParams(collective_id=0),
    )(x)


# device time: 631195 ns/iter; 1.7523x vs baseline; 1.0062x over previous
import jax
import jax.numpy as jnp
from jax import lax
from jax.experimental import pallas as pl
from jax.experimental.pallas import tpu as pltpu

N_DEV = 4
SUB = 4


def kernel(x):
    m_per, n = x.shape
    m_half = m_per // 2
    m_sub = m_half // SUB
    n_hop = N_DEV - 1

    rows = 1024
    nblk = m_per // rows

    def body(x_ref, out_ref, vbuf, ld_sems, st_sems,
             cw_send, cw_recv, ccw_send, ccw_recv):
        my = lax.axis_index("i")
        left = (my - 1) % N_DEV
        right = (my + 1) % N_DEV

        barrier = pltpu.get_barrier_semaphore()
        for nbr in (left, right):
            pl.semaphore_signal(
                barrier, inc=1,
                device_id=(nbr,), device_id_type=pl.DeviceIdType.MESH,
            )
        pl.semaphore_wait(barrier, 2)

        def make(h, q):
            o_cw = (my - h) % N_DEV
            o_ccw = (my + h) % N_DEV
            cw_rows = pl.ds(o_cw * m_per + q * m_sub, m_sub)
            ccw_rows = pl.ds(o_ccw * m_per + m_half + q * m_sub, m_sub)
            if h == 0:
                cw_src = x_ref.at[pl.ds(q * m_sub, m_sub)]
                ccw_src = x_ref.at[pl.ds(m_half + q * m_sub, m_sub)]
            else:
                cw_src = out_ref.at[cw_rows]
                ccw_src = out_ref.at[ccw_rows]
            s = h * SUB + q
            cw = pltpu.make_async_remote_copy(
                src_ref=cw_src,
                dst_ref=out_ref.at[cw_rows],
                send_sem=cw_send.at[s],
                recv_sem=cw_recv.at[s],
                device_id=(right,),
                device_id_type=pl.DeviceIdType.MESH,
            )
            ccw = pltpu.make_async_remote_copy(
                src_ref=ccw_src,
                dst_ref=out_ref.at[ccw_rows],
                send_sem=ccw_send.at[s],
                recv_sem=ccw_recv.at[s],
                device_id=(left,),
                device_id_type=pl.DeviceIdType.MESH,
            )
            return cw, ccw

        def local_copy():
            def load(b):
                return pltpu.make_async_copy(
                    x_ref.at[pl.ds(b * rows, rows)], vbuf.at[b % 2],
                    ld_sems.at[b % 2])

            def store(b):
                return pltpu.make_async_copy(
                    vbuf.at[b % 2], out_ref.at[pl.ds(my * m_per + b * rows, rows)],
                    st_sems.at[b % 2])

            load(0).start()
            for b in range(nblk):
                load(b).wait()
                store(b).start()
                if b + 1 < nblk:
                    if b >= 1:
                        store(b - 1).wait()
                    load(b + 1).start()
            store(nblk - 2).wait()
            store(nblk - 1).wait()

        rdmas = {}
        for q in range(SUB):
            cw, ccw = make(0, q)
            cw.start()
            ccw.start()
            rdmas[(0, q)] = (cw, ccw)

        local_copy()

        for h in range(1, n_hop):
            for q in range(SUB):
                prev_cw, prev_ccw = rdmas[(h - 1, q)]
                prev_cw.wait_recv()
                prev_ccw.wait_recv()
                cw, ccw = make(h, q)
                cw.start()
                ccw.start()
                rdmas[(h, q)] = (cw, ccw)

        for q in range(SUB):
            cw, ccw = rdmas[(n_hop - 1, q)]
            cw.wait_recv()
            ccw.wait_recv()
        for h in range(n_hop):
            for q in range(SUB):
                cw, ccw = rdmas[(h, q)]
                cw.wait_send()
                ccw.wait_send()

    return pl.pallas_call(
        body,
        out_shape=jax.ShapeDtypeStruct((N_DEV * m_per, n), x.dtype),
        in_specs=[pl.BlockSpec(memory_space=pl.ANY)],
        out_specs=pl.BlockSpec(memory_space=pl.ANY),
        scratch_shapes=[
            pltpu.VMEM((2, rows, n), jnp.float32),
            pltpu.SemaphoreType.DMA((2,)),
            pltpu.SemaphoreType.DMA((2,)),
            pltpu.SemaphoreType.DMA(((N_DEV - 1) * SUB,)),
            pltpu.SemaphoreType.DMA(((N_DEV - 1) * SUB,)),
            pltpu.SemaphoreType.DMA(((N_DEV - 1) * SUB,)),
            pltpu.SemaphoreType.DMA(((N_DEV - 1) * SUB,)),
        ],
        compiler_params=pltpu.CompilerParams(collective_id=0),
    )(x)
